# baseline (speedup 1.0000x reference)
import jax
import jax.numpy as jnp
from jax.experimental import pallas as pl
from jax.experimental.pallas import tpu as pltpu

_Q = 2048
_K = 16384
_D = 64
_KT = 1024
_NT = _K // _KT
_NS = _NT // 2


def _knn_body(q2_ref, qn_ref, k_ref, ks_ref, knp_ref, ksnp_ref,
              knc_ref, ksnc_ref, dis_ref, idx_ref,
              qk2a_ref, qs2a_ref, qk2b_ref, qs2b_ref, bd_ref, bi_ref):
    s = pl.program_id(0)
    q2 = q2_ref[...]
    qn = qn_ref[...]

    def dots(half, qk2_dst, qs2_dst):
        sl = pl.ds(half * _KT, _KT)
        qk2_dst[...] = jax.lax.dot_general(
            q2, k_ref[sl, :], (((1,), (1,)), ((), ())),
            preferred_element_type=jnp.float32)
        qs2_dst[...] = jax.lax.dot_general(
            q2, ks_ref[sl, :], (((1,), (1,)), ((), ())),
            preferred_element_type=jnp.float32)

    def reduce(qk2_src, qs2_src, kn_row, ksn_row, tile):
        c = jnp.minimum((qn + kn_row) + qk2_src[...],
                        (qn + ksn_row) + qs2_src[...])
        m = jnp.min(c, axis=1, keepdims=True)
        iota = jax.lax.broadcasted_iota(jnp.int32, (_Q, _KT), 1)
        a = jnp.min(jnp.where(c == m, iota, 2**30), axis=1,
                    keepdims=True) + tile * _KT
        return m, a

    mb, ab = reduce(qk2b_ref, qs2b_ref, knp_ref[0], ksnp_ref[0], 2 * s - 1)
    dots(0, qk2a_ref, qs2a_ref)

    @pl.when(s > 0)
    def _merge_b():
        bd = bd_ref[...]
        upd = mb < bd
        bd_ref[...] = jnp.where(upd, mb, bd)
        bi_ref[...] = jnp.where(upd, ab, bi_ref[...])

    ma, aa = reduce(qk2a_ref, qs2a_ref, knc_ref[0], ksnc_ref[0], 2 * s)
    dots(1, qk2b_ref, qs2b_ref)

    @pl.when(s == 0)
    def _init_a():
        bd_ref[...], bi_ref[...] = ma, aa

    @pl.when((s > 0) & (s < _NS))
    def _merge_a():
        bd = bd_ref[...]
        upd = ma < bd
        bd_ref[...] = jnp.where(upd, ma, bd)
        bi_ref[...] = jnp.where(upd, aa, bi_ref[...])

    @pl.when(s == _NS)
    def _finish():
        dis_ref[...] = bd_ref[...]
        idx_ref[...] = bi_ref[...]


def kernel(queries, keys, keys_sym):
    qn = jnp.sum(queries * queries, axis=-1, keepdims=True)
    kn = jnp.sum(keys * keys, axis=-1).reshape(_NT, 1, _KT)
    ksn = jnp.sum(keys_sym * keys_sym, axis=-1).reshape(_NT, 1, _KT)
    q2 = -2.0 * queries
    _sup = lambda s: (jnp.minimum(s, _NS - 1), 0)
    _prv = lambda s: (jnp.clip(2 * s - 1, 0, _NT - 1), 0, 0)
    _cur = lambda s: (jnp.clip(2 * s, 0, _NT - 1), 0, 0)
    dis, idx = pl.pallas_call(
        _knn_body,
        grid=(_NS + 1,),
        in_specs=[
            pl.BlockSpec((_Q, _D), lambda s: (0, 0)),
            pl.BlockSpec((_Q, 1), lambda s: (0, 0)),
            pl.BlockSpec((2 * _KT, _D), _sup),
            pl.BlockSpec((2 * _KT, _D), _sup),
            pl.BlockSpec((1, 1, _KT), _prv),
            pl.BlockSpec((1, 1, _KT), _prv),
            pl.BlockSpec((1, 1, _KT), _cur),
            pl.BlockSpec((1, 1, _KT), _cur),
        ],
        out_specs=[
            pl.BlockSpec((_Q, 1), lambda s: (0, 0)),
            pl.BlockSpec((_Q, 1), lambda s: (0, 0)),
        ],
        out_shape=[
            jax.ShapeDtypeStruct((_Q, 1), jnp.float32),
            jax.ShapeDtypeStruct((_Q, 1), jnp.int32),
        ],
        scratch_shapes=[
            pltpu.VMEM((_Q, _KT), jnp.float32),
            pltpu.VMEM((_Q, _KT), jnp.float32),
            pltpu.VMEM((_Q, _KT), jnp.float32),
            pltpu.VMEM((_Q, _KT), jnp.float32),
            pltpu.VMEM((_Q, 1), jnp.float32),
            pltpu.VMEM((_Q, 1), jnp.int32),
        ],
        compiler_params=pltpu.CompilerParams(
            dimension_semantics=("arbitrary",)),
    )(q2, qn, keys, keys_sym, kn, ksn, kn, ksn)
    return dis[:, 0], idx[:, 0]


# speedup vs baseline: 1.0316x; 1.0316x over previous
import jax
import jax.numpy as jnp
from jax.experimental import pallas as pl
from jax.experimental.pallas import tpu as pltpu

_Q = 2048
_K = 16384
_D = 64
_KT = 1024
_NT = _K // _KT
_NS = _NT // 2


def _knn_body(q2_ref, qn_ref, k_ref, ks_ref, knp_ref, ksnp_ref,
              knc_ref, ksnc_ref, fiota_ref, dis_ref, idx_ref,
              qk2a_ref, qs2a_ref, qk2b_ref, qs2b_ref, bd_ref, bi_ref):
    s = pl.program_id(0)
    q2 = q2_ref[...]
    qn = qn_ref[...]

    def dots(half, qk2_dst, qs2_dst):
        sl = pl.ds(half * _KT, _KT)
        qk2_dst[...] = jax.lax.dot_general(
            q2, k_ref[sl, :], (((1,), (1,)), ((), ())),
            preferred_element_type=jnp.float32)
        qs2_dst[...] = jax.lax.dot_general(
            q2, ks_ref[sl, :], (((1,), (1,)), ((), ())),
            preferred_element_type=jnp.float32)

    def reduce(qk2_src, qs2_src, kn_row, ksn_row, tile):
        c = jnp.minimum((qn + kn_row) + qk2_src[...],
                        (qn + ksn_row) + qs2_src[...])
        m = jnp.min(c, axis=1, keepdims=True)
        af = jnp.min(jnp.where(c == m, fiota_ref[0], jnp.float32(2**30)),
                     axis=1, keepdims=True)
        a = af.astype(jnp.int32) + tile * _KT
        return m, a

    mb, ab = reduce(qk2b_ref, qs2b_ref, knp_ref[0], ksnp_ref[0], 2 * s - 1)
    dots(0, qk2a_ref, qs2a_ref)

    @pl.when(s > 0)
    def _merge_b():
        bd = bd_ref[...]
        upd = mb < bd
        bd_ref[...] = jnp.where(upd, mb, bd)
        bi_ref[...] = jnp.where(upd, ab, bi_ref[...])

    ma, aa = reduce(qk2a_ref, qs2a_ref, knc_ref[0], ksnc_ref[0], 2 * s)
    dots(1, qk2b_ref, qs2b_ref)

    @pl.when(s == 0)
    def _init_a():
        bd_ref[...], bi_ref[...] = ma, aa

    @pl.when((s > 0) & (s < _NS))
    def _merge_a():
        bd = bd_ref[...]
        upd = ma < bd
        bd_ref[...] = jnp.where(upd, ma, bd)
        bi_ref[...] = jnp.where(upd, aa, bi_ref[...])

    @pl.when(s == _NS)
    def _finish():
        dis_ref[...] = bd_ref[...]
        idx_ref[...] = bi_ref[...]


def kernel(queries, keys, keys_sym):
    qn = jnp.sum(queries * queries, axis=-1, keepdims=True)
    kn = jnp.sum(keys * keys, axis=-1).reshape(_NT, 1, _KT)
    ksn = jnp.sum(keys_sym * keys_sym, axis=-1).reshape(_NT, 1, _KT)
    q2 = -2.0 * queries
    _sup = lambda s: (jnp.minimum(s, _NS - 1), 0)
    _prv = lambda s: (jnp.clip(2 * s - 1, 0, _NT - 1), 0, 0)
    _cur = lambda s: (jnp.clip(2 * s, 0, _NT - 1), 0, 0)
    dis, idx = pl.pallas_call(
        _knn_body,
        grid=(_NS + 1,),
        in_specs=[
            pl.BlockSpec((_Q, _D), lambda s: (0, 0)),
            pl.BlockSpec((_Q, 1), lambda s: (0, 0)),
            pl.BlockSpec((2 * _KT, _D), _sup),
            pl.BlockSpec((2 * _KT, _D), _sup),
            pl.BlockSpec((1, 1, _KT), _prv),
            pl.BlockSpec((1, 1, _KT), _prv),
            pl.BlockSpec((1, 1, _KT), _cur),
            pl.BlockSpec((1, 1, _KT), _cur),
            pl.BlockSpec((1, 1, _KT), lambda s: (0, 0, 0)),
        ],
        out_specs=[
            pl.BlockSpec((_Q, 1), lambda s: (0, 0)),
            pl.BlockSpec((_Q, 1), lambda s: (0, 0)),
        ],
        out_shape=[
            jax.ShapeDtypeStruct((_Q, 1), jnp.float32),
            jax.ShapeDtypeStruct((_Q, 1), jnp.int32),
        ],
        scratch_shapes=[
            pltpu.VMEM((_Q, _KT), jnp.float32),
            pltpu.VMEM((_Q, _KT), jnp.float32),
            pltpu.VMEM((_Q, _KT), jnp.float32),
            pltpu.VMEM((_Q, _KT), jnp.float32),
            pltpu.VMEM((_Q, 1), jnp.float32),
            pltpu.VMEM((_Q, 1), jnp.int32),
        ],
        compiler_params=pltpu.CompilerParams(
            dimension_semantics=("arbitrary",)),
    )(q2, qn, keys, keys_sym, kn, ksn, kn, ksn,
      jnp.arange(_KT, dtype=jnp.float32).reshape(1, 1, _KT))
    return dis[:, 0], idx[:, 0]
